# baseline (speedup 1.0000x reference)
import functools

import jax
import jax.numpy as jnp
from jax import lax
from jax.experimental import pallas as pl
from jax.experimental.pallas import tpu as pltpu
from jax.experimental.pallas import tpu_sc as plsc


def _sc_gather_one(EP, idx3):
    NW, NCH, CH = idx3.shape
    B = NW * NCH * CH
    W = EP.shape[1]
    info = plsc.get_sparse_core_info()
    NC = info.num_cores
    bpw = B // NW

    mesh = plsc.VectorSubcoreMesh(core_axis_name="c", subcore_axis_name="s")

    @functools.partial(
        pl.kernel,
        mesh=mesh,
        out_type=jax.ShapeDtypeStruct((B, W), jnp.float32),
        scratch_types=[
            pltpu.VMEM((NCH, CH), jnp.int32),
            pltpu.VMEM((bpw, W), jnp.float32),
            pltpu.SemaphoreType.DMA,
        ],
    )
    def gather_kernel(ep_hbm, i_hbm, o_hbm, idxv, rows, sem):
        wid = lax.axis_index("s") * NC + lax.axis_index("c")
        base = wid * bpw
        pltpu.sync_copy(i_hbm.at[wid], idxv)
        cps = [
            pltpu.async_copy(ep_hbm.at[idxv.at[c]],
                             rows.at[pl.ds(c * CH, CH)], sem)
            for c in range(NCH)
        ]
        for cp in cps:
            cp.wait()
        pltpu.sync_copy(rows, o_hbm.at[pl.ds(base, bpw)])

    return gather_kernel(EP, idx3)


_RW = 16384


def _tc_relayout_pairs(ET):
    Dd, V = ET.shape
    G = pl.cdiv(V, _RW)

    def body(et_ref, o_ref):
        blk = et_ref[...]
        t = jax.lax.transpose(blk, (1, 0))
        o_ref[...] = jnp.concatenate(
            [t[:_RW // 2], t[_RW // 2:]], axis=1)

    return pl.pallas_call(
        body,
        grid=(G,),
        in_specs=[pl.BlockSpec((Dd, _RW), lambda i: (0, i))],
        out_specs=pl.BlockSpec((_RW // 2, 2 * Dd), lambda i: (i, 0)),
        out_shape=jax.ShapeDtypeStruct((G * (_RW // 2), 2 * Dd), jnp.float32),
        compiler_params=pltpu.CompilerParams(
            dimension_semantics=("arbitrary",),
        ),
    )(ET)


def _tc_dense_assemble(uf, itf, WuT, WiT, bu, bi, pu, pi_, upar, ipar):
    B, F = uf.shape
    D = WuT.shape[1]
    bB = 2048

    def body(uf_ref, if_ref, wu_ref, wi_ref, bu_ref, bi_ref,
             pu_ref, pi_ref, up_ref, ip_ref, o_ref):
        du = jnp.dot(uf_ref[...], wu_ref[...],
                     preferred_element_type=jnp.float32) + bu_ref[...]
        di = jnp.dot(if_ref[...], wi_ref[...],
                     preferred_element_type=jnp.float32) + bi_ref[...]
        eu = jnp.where(up_ref[...] == 1, pu_ref[:, D:2 * D], pu_ref[:, :D])
        ei = jnp.where(ip_ref[...] == 1, pi_ref[:, D:2 * D], pi_ref[:, :D])
        o_ref[...] = jnp.concatenate([du, di, eu, ei], axis=-1)

    out = pl.pallas_call(
        body,
        grid=(B // bB,),
        in_specs=[
            pl.BlockSpec((bB, F), lambda i: (i, 0)),
            pl.BlockSpec((bB, F), lambda i: (i, 0)),
            pl.BlockSpec((F, D), lambda i: (0, 0)),
            pl.BlockSpec((F, D), lambda i: (0, 0)),
            pl.BlockSpec((1, D), lambda i: (0, 0)),
            pl.BlockSpec((1, D), lambda i: (0, 0)),
            pl.BlockSpec((bB, 2 * D), lambda i: (i, 0)),
            pl.BlockSpec((bB, 2 * D), lambda i: (i, 0)),
            pl.BlockSpec((bB, 1), lambda i: (i, 0)),
            pl.BlockSpec((bB, 1), lambda i: (i, 0)),
        ],
        out_specs=pl.BlockSpec((bB, 4 * D), lambda i: (i, 0)),
        out_shape=jax.ShapeDtypeStruct((B, 4 * D), jnp.float32),
        compiler_params=pltpu.CompilerParams(
            dimension_semantics=("arbitrary",),
        ),
    )(uf, itf, WuT, WiT, bu, bi, pu, pi_, upar, ipar)
    return out


def kernel(users_features, items_features, user, item,
           W_user, b_user, W_item, b_item, E_user, E_item):
    B = users_features.shape[0]
    D = W_user.shape[0]
    V = E_user.shape[0]

    NW, CH = 32, 128
    NCH = B // (NW * CH)
    user = user.astype(jnp.int32)
    item = item.astype(jnp.int32)
    EuP = _tc_relayout_pairs(E_user.T)
    HW = _RW // 2
    user_pair = (user // _RW) * HW + user % HW
    upar = ((user % _RW) >= HW).astype(jnp.int32).reshape(B, 1)

    EiP = E_item.reshape(V // 2, 2 * D)
    item_pair = item >> 1
    ipar = (item & 1).reshape(B, 1)

    user_p3 = user_pair.reshape(NW, NCH, CH)
    item_p3 = item_pair.reshape(NW, NCH, CH)
    pu = _sc_gather_one(EuP, user_p3)
    pi_ = _sc_gather_one(EiP, item_p3)
    out = _tc_dense_assemble(
        users_features, items_features,
        W_user.T, W_item.T,
        b_user.reshape(1, D), b_item.reshape(1, D),
        pu, pi_, upar, ipar,
    )
    return out.reshape(B, 4, D)


# speedup vs baseline: 1.6163x; 1.6163x over previous
import functools

import jax
import jax.numpy as jnp
from jax import lax
from jax.experimental import pallas as pl
from jax.experimental.pallas import tpu as pltpu
from jax.experimental.pallas import tpu_sc as plsc


def _sc_gather_one(EP, idx3):
    NW, NCH, CH = idx3.shape
    B = NW * NCH * CH
    W = EP.shape[1]
    info = plsc.get_sparse_core_info()
    NC = info.num_cores
    bpw = B // NW

    mesh = plsc.VectorSubcoreMesh(core_axis_name="c", subcore_axis_name="s")

    @functools.partial(
        pl.kernel,
        mesh=mesh,
        out_type=jax.ShapeDtypeStruct((B, W), jnp.float32),
        scratch_types=[
            pltpu.VMEM((NCH, CH), jnp.int32),
            pltpu.VMEM((bpw, W), jnp.float32),
            pltpu.SemaphoreType.DMA,
        ],
    )
    def gather_kernel(ep_hbm, i_hbm, o_hbm, idxv, rows, sem):
        wid = lax.axis_index("s") * NC + lax.axis_index("c")
        base = wid * bpw
        pltpu.sync_copy(i_hbm.at[wid], idxv)
        cps = [
            pltpu.async_copy(ep_hbm.at[idxv.at[c]],
                             rows.at[pl.ds(c * CH, CH)], sem)
            for c in range(NCH)
        ]
        for cp in cps:
            cp.wait()
        pltpu.sync_copy(rows, o_hbm.at[pl.ds(base, bpw)])

    return gather_kernel(EP, idx3)


_RW = 32768


def _tc_relayout_pairs(ET):
    Dd, V = ET.shape
    G = pl.cdiv(V, _RW)

    def body(et_ref, o_ref):
        blk = et_ref[...]
        t = jax.lax.transpose(blk, (1, 0))
        o_ref[...] = jnp.concatenate(
            [t[:_RW // 2], t[_RW // 2:]], axis=1)

    return pl.pallas_call(
        body,
        grid=(G,),
        in_specs=[pl.BlockSpec((Dd, _RW), lambda i: (0, i))],
        out_specs=pl.BlockSpec((_RW // 2, 2 * Dd), lambda i: (i, 0)),
        out_shape=jax.ShapeDtypeStruct((G * (_RW // 2), 2 * Dd), jnp.float32),
        compiler_params=pltpu.CompilerParams(
            dimension_semantics=("arbitrary",),
        ),
    )(ET)


def _tc_dense_assemble(uf, itf, WuT, WiT, bu, bi, pu, pi_, upar, ipar):
    B, F = uf.shape
    D = WuT.shape[1]
    bB = 2048

    def body(uf_ref, if_ref, wu_ref, wi_ref, bu_ref, bi_ref,
             pu_ref, pi_ref, up_ref, ip_ref, o_ref):
        du = jnp.dot(uf_ref[...], wu_ref[...],
                     preferred_element_type=jnp.float32) + bu_ref[...]
        di = jnp.dot(if_ref[...], wi_ref[...],
                     preferred_element_type=jnp.float32) + bi_ref[...]
        eu = jnp.where(up_ref[...] == 1, pu_ref[:, D:2 * D], pu_ref[:, :D])
        ei = jnp.where(ip_ref[...] == 1, pi_ref[:, D:2 * D], pi_ref[:, :D])
        o_ref[...] = jnp.concatenate([du, di, eu, ei], axis=-1)

    out = pl.pallas_call(
        body,
        grid=(B // bB,),
        in_specs=[
            pl.BlockSpec((bB, F), lambda i: (i, 0)),
            pl.BlockSpec((bB, F), lambda i: (i, 0)),
            pl.BlockSpec((F, D), lambda i: (0, 0)),
            pl.BlockSpec((F, D), lambda i: (0, 0)),
            pl.BlockSpec((1, D), lambda i: (0, 0)),
            pl.BlockSpec((1, D), lambda i: (0, 0)),
            pl.BlockSpec((bB, 2 * D), lambda i: (i, 0)),
            pl.BlockSpec((bB, 2 * D), lambda i: (i, 0)),
            pl.BlockSpec((bB, 1), lambda i: (i, 0)),
            pl.BlockSpec((bB, 1), lambda i: (i, 0)),
        ],
        out_specs=pl.BlockSpec((bB, 4 * D), lambda i: (i, 0)),
        out_shape=jax.ShapeDtypeStruct((B, 4 * D), jnp.float32),
        compiler_params=pltpu.CompilerParams(
            dimension_semantics=("arbitrary",),
        ),
    )(uf, itf, WuT, WiT, bu, bi, pu, pi_, upar, ipar)
    return out


def kernel(users_features, items_features, user, item,
           W_user, b_user, W_item, b_item, E_user, E_item):
    B = users_features.shape[0]
    D = W_user.shape[0]
    V = E_user.shape[0]

    NW, CH = 32, 128
    NCH = B // (NW * CH)
    user = user.astype(jnp.int32)
    item = item.astype(jnp.int32)
    EuP = _tc_relayout_pairs(E_user.T)
    EiP = _tc_relayout_pairs(E_item.T)
    HW = _RW // 2
    user_pair = (user // _RW) * HW + user % HW
    item_pair = (item // _RW) * HW + item % HW
    upar = ((user % _RW) >= HW).astype(jnp.int32).reshape(B, 1)
    ipar = ((item % _RW) >= HW).astype(jnp.int32).reshape(B, 1)

    user_p3 = user_pair.reshape(NW, NCH, CH)
    item_p3 = item_pair.reshape(NW, NCH, CH)
    pu = _sc_gather_one(EuP, user_p3)
    pi_ = _sc_gather_one(EiP, item_p3)
    out = _tc_dense_assemble(
        users_features, items_features,
        W_user.T, W_item.T,
        b_user.reshape(1, D), b_item.reshape(1, D),
        pu, pi_, upar, ipar,
    )
    return out.reshape(B, 4, D)
